# baseline (speedup 1.0000x reference)
import jax
import jax.numpy as jnp
from jax.experimental import pallas as pl

_CC, _NN = 64, 256


def _spatial_conv_kernel(y3_ref, infos_ref, wd_ref, bd_ref, wa_ref, ba_ref,
                         out_ref):
    C, N = _CC, _NN
    f = pl.program_id(1)
    W_src = wd_ref[:, :C] + wd_ref[:, 2 * C:]
    W_dst = wd_ref[:, C:2 * C] - wd_ref[:, 2 * C:]

    G = infos_ref[0, :, pl.ds(f * N, N)]
    Az = y3_ref[0, 0] != 0
    A = (Az | Az.T).astype(jnp.float32)
    deg = jnp.sum(A, axis=1)
    rdeg = 1.0 / jnp.maximum(deg, 1.0)

    Psrc = jnp.dot(W_src, G, preferred_element_type=jnp.float32)
    M1 = G * (Psrc + bd_ref[0, :][:, None])
    Pdst = jnp.dot(W_dst, G, preferred_element_type=jnp.float32)

    X = jnp.concatenate([M1, G], axis=0)
    R = jnp.dot(X, A, preferred_element_type=jnp.float32)
    red = (R[:C] + R[C:] * Pdst) * rdeg[None, :]
    red = jnp.where((deg > 0.0)[None, :], red, G)
    applied = jnp.dot(wa_ref[...], red, preferred_element_type=jnp.float32)
    out_ref[0, :, pl.ds(f * N, N)] = jax.nn.relu(
        applied + ba_ref[0, :][:, None])


def _group_of(f):
    return (f > 0).astype(jnp.int32) + (f > 11).astype(jnp.int32)


@jax.jit
def kernel(Y, infos, W_dense, b_dense, W_apply, b_apply):
    B, C, F, N = infos.shape
    Y3 = Y[:, :3]
    out = pl.pallas_call(
        _spatial_conv_kernel,
        grid=(B, F),
        in_specs=[
            pl.BlockSpec((1, 1, N, N), lambda b, f: (b, _group_of(f), 0, 0)),
            pl.BlockSpec((1, C, F * N), lambda b, f: (b, 0, 0)),
            pl.BlockSpec((C, 3 * C), lambda b, f: (0, 0)),
            pl.BlockSpec((1, C), lambda b, f: (0, 0)),
            pl.BlockSpec((C, C), lambda b, f: (0, 0)),
            pl.BlockSpec((1, C), lambda b, f: (0, 0)),
        ],
        out_specs=pl.BlockSpec((1, C, F * N), lambda b, f: (b, 0, 0)),
        out_shape=jax.ShapeDtypeStruct((B, C, F * N), jnp.float32),
    )(Y3, infos.reshape(B, C, F * N), W_dense, b_dense.reshape(1, C),
      W_apply, b_apply.reshape(1, C))
    return out.reshape(B, C, F, N)
